# Initial kernel scaffold; baseline (speedup 1.0000x reference)
import functools
import math

import jax
import jax.numpy as jnp
from jax import lax
from jax.experimental import pallas as pl
from jax.experimental.pallas import tpu as pltpu
from jax.experimental.pallas import tpu_sc as plsc

N_NODES = 100000
N_CH = 4
SIGMA = 0.2
INV_SQRT2_SIGMA = 1.0 / (math.sqrt(2.0) * SIGMA)

NC = 2
NS = 16
NW = NC * NS

B = 2048
CH = B // 128

P_C = 0.3275911
A1 = 0.254829592
A2 = -0.284496736
A3 = 1.421413741
A4 = -1.453152027
A5 = 1.061405429

ROWS_PER_TILE = N_NODES // NS


def _sc_body(q_hbm, ii_hbm, jj_hbm, d_hbm, z_hbm, out_hbm,
             d_buf, pot_buf, ii_buf, jj_buf, rows_i, rows_j, q_sp, acc_sp,
             *, n_blocks):
    c = lax.axis_index("c")
    s = lax.axis_index("s")
    tile = c * NS + s

    r0 = s * ROWS_PER_TILE
    pltpu.sync_copy(q_hbm.at[pl.ds(r0, ROWS_PER_TILE)],
                    q_sp.at[pl.ds(r0, ROWS_PER_TILE)])
    pltpu.sync_copy(z_hbm.at[pl.ds(r0, ROWS_PER_TILE)],
                    acc_sp.at[pl.ds(r0, ROWS_PER_TILE)])
    plsc.subcore_barrier()

    iota16 = lax.iota(jnp.int32, 16)
    iota4 = iota16 // 4
    iotac = iota16 % 4

    edges_per_tile = n_blocks * B
    chunk0 = tile * (edges_per_tile // 128)

    def block_fn(b, _):
        base_e = tile * edges_per_tile + b * B
        base_ck = chunk0 + b * CH
        pltpu.sync_copy(d_hbm.at[pl.ds(base_e, B)], d_buf)
        pltpu.sync_copy(ii_hbm.at[pl.ds(base_ck, CH)], ii_buf)
        pltpu.sync_copy(jj_hbm.at[pl.ds(base_ck, CH)], jj_buf)

        def pot_fn(k, _):
            d = d_buf[pl.ds(k * 16, 16)]
            x = d * INV_SQRT2_SIGMA
            t = 1.0 / (1.0 + P_C * x)
            poly = t * (A1 + t * (A2 + t * (A3 + t * (A4 + t * A5))))
            pot_buf[pl.ds(k * 16, 16)] = 0.5 * poly * jnp.exp(-x * x) / d
            return 0
        lax.fori_loop(0, B // 16, pot_fn, 0, unroll=4)

        pltpu.sync_copy(q_sp.at[ii_buf], rows_i)
        pltpu.sync_copy(q_sp.at[jj_buf], rows_j)

        def mul_fn(v, _):
            k = v // 32
            e16 = (v % 32) * 4 + iota4
            pe = v * 4 + iota4
            p = plsc.load_gather(pot_buf, [pe])
            rj = plsc.load_gather(rows_j.at[k], [e16, iotac])
            plsc.store_scatter(rows_j.at[k], [e16, iotac], rj * p)
            ri = plsc.load_gather(rows_i.at[k], [e16, iotac])
            plsc.store_scatter(rows_i.at[k], [e16, iotac], ri * p)
            return 0
        lax.fori_loop(0, (B * N_CH) // 16, mul_fn, 0, unroll=4)

        pltpu.sync_copy(rows_j, acc_sp.at[ii_buf], add=True)
        pltpu.sync_copy(rows_i, acc_sp.at[jj_buf], add=True)
        return 0

    lax.fori_loop(0, n_blocks, block_fn, 0)

    plsc.subcore_barrier()
    pltpu.sync_copy(acc_sp.at[pl.ds(r0, ROWS_PER_TILE)],
                    out_hbm.at[c].at[pl.ds(r0, ROWS_PER_TILE)])


def _combine_body(p_ref, o_ref):
    o_ref[...] = p_ref[0] + p_ref[1]


def kernel(smearing, charges, neighbor_indices, neighbor_distances):
    del smearing
    e_total = neighbor_distances.shape[0]
    n_blocks = -(-e_total // (NW * B))
    e_pad = n_blocks * NW * B
    pad = e_pad - e_total

    idx = neighbor_indices.astype(jnp.int32)
    ii = jnp.concatenate([idx[:, 0], jnp.zeros((pad,), jnp.int32)])
    jj = jnp.concatenate([idx[:, 1], jnp.zeros((pad,), jnp.int32)])
    dist = jnp.concatenate(
        [neighbor_distances.astype(jnp.float32),
         jnp.full((pad,), 1e9, jnp.float32)])
    ii2 = ii.reshape(e_pad // 128, 128)
    jj2 = jj.reshape(e_pad // 128, 128)
    q = charges.astype(jnp.float32)
    zeros = jnp.zeros((N_NODES, N_CH), jnp.float32)

    mesh = plsc.VectorSubcoreMesh(core_axis_name="c", subcore_axis_name="s")
    sc_call = pl.kernel(
        functools.partial(_sc_body, n_blocks=n_blocks),
        out_type=jax.ShapeDtypeStruct((NC, N_NODES, N_CH), jnp.float32),
        mesh=mesh,
        scratch_types=[
            pltpu.VMEM((B,), jnp.float32),
            pltpu.VMEM((B,), jnp.float32),
            pltpu.VMEM((CH, 128), jnp.int32),
            pltpu.VMEM((CH, 128), jnp.int32),
            pltpu.VMEM((CH, 128, N_CH), jnp.float32),
            pltpu.VMEM((CH, 128, N_CH), jnp.float32),
            pltpu.VMEM_SHARED((N_NODES, N_CH), jnp.float32),
            pltpu.VMEM_SHARED((N_NODES, N_CH), jnp.float32),
        ],
    )
    partials = sc_call(q, ii2, jj2, dist, zeros)

    flat = partials.reshape(NC, (N_NODES * N_CH) // 128, 128)
    out = pl.pallas_call(
        _combine_body,
        out_shape=jax.ShapeDtypeStruct(((N_NODES * N_CH) // 128, 128),
                                       jnp.float32),
    )(flat)
    return out.reshape(N_NODES, N_CH)


# speedup vs baseline: 50.5804x; 50.5804x over previous
import functools
import math

import jax
import jax.numpy as jnp
from jax import lax
from jax.experimental import pallas as pl
from jax.experimental.pallas import tpu as pltpu
from jax.experimental.pallas import tpu_sc as plsc

N_NODES = 100000
N_CH = 4
D_PAD = 8
N_PAD = 100352
SIGMA = 0.2
INV_SQRT2_SIGMA = 1.0 / (math.sqrt(2.0) * SIGMA)

NC = 2
NS = 16
NW = NC * NS

B = 2048

P_C = 0.3275911
A1 = 0.254829592
A2 = -0.284496736
A3 = 1.421413741
A4 = -1.453152027
A5 = 1.061405429

ROWS_PER_TILE = N_PAD // NS


def _sc_body(q_hbm, ii_hbm, jj_hbm, d_hbm, out_hbm,
             d0, ii0, jj0, ri0, rj0,
             d1, ii1, jj1, ri1, rj1,
             pot_buf, tab_sp,
             sem_l0, sem_l1, sem_g, sem_s0, sem_s1,
             *, n_blocks):
    c = lax.axis_index("c")
    s = lax.axis_index("s")
    tile = c * NS + s

    r0 = s * ROWS_PER_TILE
    pltpu.sync_copy(q_hbm.at[pl.ds(r0, ROWS_PER_TILE)],
                    tab_sp.at[pl.ds(r0, ROWS_PER_TILE)])
    plsc.subcore_barrier()

    iota16 = lax.iota(jnp.int32, 16)
    iota4 = iota16 // 4
    iotac = iota16 % 4
    zero16 = jnp.zeros((16,), jnp.float32)

    edges_per_tile = n_blocks * B

    sets = ((d0, ii0, jj0, ri0, rj0, sem_l0, sem_s0),
            (d1, ii1, jj1, ri1, rj1, sem_l1, sem_s1))

    def fire_lin(b, st):
        base = tile * edges_per_tile + b * B
        pltpu.async_copy(d_hbm.at[pl.ds(base, B)], st[0], st[5])
        pltpu.async_copy(ii_hbm.at[pl.ds(base, B)], st[1], st[5])
        pltpu.async_copy(jj_hbm.at[pl.ds(base, B)], st[2], st[5])

    def drain_lin(b, st):
        base = tile * edges_per_tile + b * B
        pltpu.make_async_copy(d_hbm.at[pl.ds(base, B)], st[0], st[5]).wait()
        pltpu.make_async_copy(ii_hbm.at[pl.ds(base, B)], st[1], st[5]).wait()
        pltpu.make_async_copy(jj_hbm.at[pl.ds(base, B)], st[2], st[5]).wait()

    def drain_scat(st):
        pltpu.make_async_copy(q_hbm.at[pl.ds(0, B)], st[3], st[6]).wait()
        pltpu.make_async_copy(q_hbm.at[pl.ds(0, B)], st[4], st[6]).wait()

    def process(b, st, oth, wait_prev, fire_next):
        d_b, ii_b, jj_b, ri, rj, _, sem_s = st
        drain_lin(b, st)

        gat = [pltpu.async_copy(tab_sp.at[ii_b], ri, sem_g),
               pltpu.async_copy(tab_sp.at[jj_b], rj, sem_g)]

        def pot_fn(k, _):
            d = d_b[pl.ds(k * 16, 16)]
            x = d * INV_SQRT2_SIGMA
            t = 1.0 / (1.0 + P_C * x)
            poly = t * (A1 + t * (A2 + t * (A3 + t * (A4 + t * A5))))
            pot_buf[pl.ds(k * 16, 16)] = 0.5 * poly * jnp.exp(-x * x) / d
            return 0
        lax.fori_loop(0, B // 16, pot_fn, 0, unroll=4)

        if wait_prev:
            drain_scat(oth)
        if fire_next:
            fire_lin(b + 1, oth)

        for cp_ in gat:
            cp_.wait()

        def mul_fn(v, _):
            pe = v * 4 + iota4
            p = plsc.load_gather(pot_buf, [pe])
            vj = plsc.load_gather(rj, [pe, iotac])
            plsc.store_scatter(rj, [pe, iotac + 4], vj * p)
            plsc.store_scatter(rj, [pe, iotac], zero16)
            vi = plsc.load_gather(ri, [pe, iotac])
            plsc.store_scatter(ri, [pe, iotac + 4], vi * p)
            plsc.store_scatter(ri, [pe, iotac], zero16)
            return 0
        lax.fori_loop(0, (B * N_CH) // 16, mul_fn, 0, unroll=4)

        pltpu.async_copy(rj, tab_sp.at[ii_b], sem_s, add=True)
        pltpu.async_copy(ri, tab_sp.at[jj_b], sem_s, add=True)

    fire_lin(0, sets[0])
    fire_lin(1, sets[1])
    process(0, sets[0], sets[1], False, False)
    if n_blocks >= 4:
        process(1, sets[1], sets[0], True, True)

        def pair_fn(k, _):
            b = 2 * k
            process(b, sets[0], sets[1], True, True)
            process(b + 1, sets[1], sets[0], True, True)
            return 0
        lax.fori_loop(1, n_blocks // 2 - 1, pair_fn, 0)

        process(n_blocks - 2, sets[0], sets[1], True, True)
        process(n_blocks - 1, sets[1], sets[0], True, False)
    else:
        process(1, sets[1], sets[0], True, False)
    drain_scat(sets[1])

    plsc.subcore_barrier()
    pltpu.sync_copy(tab_sp.at[pl.ds(r0, ROWS_PER_TILE)],
                    out_hbm.at[c].at[pl.ds(r0, ROWS_PER_TILE)])


def _combine_body(p_ref, o_ref):
    o_ref[...] = p_ref[0] + p_ref[1]


def kernel(smearing, charges, neighbor_indices, neighbor_distances):
    del smearing
    e_total = neighbor_distances.shape[0]
    n_pairs = -(-e_total // (NW * B * 2))
    n_blocks = 2 * n_pairs
    e_pad = n_blocks * NW * B
    pad = e_pad - e_total

    idx = neighbor_indices.astype(jnp.int32)
    ii = jnp.concatenate([idx[:, 0], jnp.zeros((pad,), jnp.int32)])
    jj = jnp.concatenate([idx[:, 1], jnp.zeros((pad,), jnp.int32)])
    dist = jnp.concatenate(
        [neighbor_distances.astype(jnp.float32),
         jnp.full((pad,), 1e9, jnp.float32)])
    q = jnp.zeros((N_PAD, D_PAD), jnp.float32)
    q = q.at[:N_NODES, :N_CH].set(charges.astype(jnp.float32))

    mesh = plsc.VectorSubcoreMesh(core_axis_name="c", subcore_axis_name="s",
                                  num_cores=NC, num_subcores=NS)
    sc_call = pl.kernel(
        functools.partial(_sc_body, n_blocks=n_blocks),
        out_type=jax.ShapeDtypeStruct((NC, N_PAD, D_PAD), jnp.float32),
        mesh=mesh,
        compiler_params=pltpu.CompilerParams(use_tc_tiling_on_sc=False,
                                             needs_layout_passes=False),
        scratch_types=[
            pltpu.VMEM((B,), jnp.float32),
            pltpu.VMEM((B,), jnp.int32),
            pltpu.VMEM((B,), jnp.int32),
            pltpu.VMEM((B, D_PAD), jnp.float32),
            pltpu.VMEM((B, D_PAD), jnp.float32),
            pltpu.VMEM((B,), jnp.float32),
            pltpu.VMEM((B,), jnp.int32),
            pltpu.VMEM((B,), jnp.int32),
            pltpu.VMEM((B, D_PAD), jnp.float32),
            pltpu.VMEM((B, D_PAD), jnp.float32),
            pltpu.VMEM((B,), jnp.float32),
            pltpu.VMEM_SHARED((N_PAD, D_PAD), jnp.float32),
            pltpu.SemaphoreType.DMA,
            pltpu.SemaphoreType.DMA,
            pltpu.SemaphoreType.DMA,
            pltpu.SemaphoreType.DMA,
            pltpu.SemaphoreType.DMA,
        ],
    )
    partials = sc_call(q, ii, jj, dist)

    flat = partials.reshape(NC, (N_PAD * D_PAD) // 128, 128)
    out = pl.pallas_call(
        _combine_body,
        out_shape=jax.ShapeDtypeStruct(((N_PAD * D_PAD) // 128, 128),
                                       jnp.float32),
    )(flat)
    return out.reshape(N_PAD, D_PAD)[:N_NODES, N_CH:]


# speedup vs baseline: 54.2142x; 1.0718x over previous
import functools
import math

import jax
import jax.numpy as jnp
from jax import lax
from jax.experimental import pallas as pl
from jax.experimental.pallas import tpu as pltpu
from jax.experimental.pallas import tpu_sc as plsc

N_NODES = 100000
N_CH = 4
D_PAD = 8
N_PAD = 100352
SIGMA = 0.2
INV_SQRT2_SIGMA = 1.0 / (math.sqrt(2.0) * SIGMA)

NC = 2
NS = 16
NW = NC * NS

B = 1344
NBUF = 3

P_C = 0.3275911
A1 = 0.254829592
A2 = -0.284496736
A3 = 1.421413741
A4 = -1.453152027
A5 = 1.061405429

ROWS_PER_TILE = N_PAD // NS


def _sc_body(q_hbm, ii_hbm, jj_hbm, d_hbm, out_hbm,
             d0, ii0, jj0, ri0, rj0,
             d1, ii1, jj1, ri1, rj1,
             d2, ii2, jj2, ri2, rj2,
             pot_buf, tab_sp,
             sem_l0, sem_l1, sem_l2,
             sem_g0, sem_g1, sem_g2,
             sem_s0, sem_s1, sem_s2,
             *, n_blocks):
    c = lax.axis_index("c")
    s = lax.axis_index("s")
    tile = c * NS + s

    r0 = s * ROWS_PER_TILE
    pltpu.sync_copy(q_hbm.at[pl.ds(r0, ROWS_PER_TILE)],
                    tab_sp.at[pl.ds(r0, ROWS_PER_TILE)])
    plsc.subcore_barrier()

    iota16 = lax.iota(jnp.int32, 16)
    iota4 = iota16 // 4
    iotac = iota16 % 4
    zero16 = jnp.zeros((16,), jnp.float32)

    edges_per_tile = n_blocks * B

    sets = ((d0, ii0, jj0, ri0, rj0, sem_l0, sem_g0, sem_s0),
            (d1, ii1, jj1, ri1, rj1, sem_l1, sem_g1, sem_s1),
            (d2, ii2, jj2, ri2, rj2, sem_l2, sem_g2, sem_s2))

    def fire_lin(b, st):
        base = tile * edges_per_tile + b * B
        pltpu.async_copy(d_hbm.at[pl.ds(base, B)], st[0], st[5])
        pltpu.async_copy(ii_hbm.at[pl.ds(base, B)], st[1], st[5])
        pltpu.async_copy(jj_hbm.at[pl.ds(base, B)], st[2], st[5])

    def drain_lin(b, st):
        base = tile * edges_per_tile + b * B
        pltpu.make_async_copy(d_hbm.at[pl.ds(base, B)], st[0], st[5]).wait()
        pltpu.make_async_copy(ii_hbm.at[pl.ds(base, B)], st[1], st[5]).wait()
        pltpu.make_async_copy(jj_hbm.at[pl.ds(base, B)], st[2], st[5]).wait()

    def fire_gather(st):
        pltpu.async_copy(tab_sp.at[st[1]], st[3], st[6])
        pltpu.async_copy(tab_sp.at[st[2]], st[4], st[6])

    def drain_gather(st):
        pltpu.make_async_copy(q_hbm.at[pl.ds(0, B)], st[3], st[6]).wait()
        pltpu.make_async_copy(q_hbm.at[pl.ds(0, B)], st[4], st[6]).wait()

    def drain_scat(st):
        pltpu.make_async_copy(q_hbm.at[pl.ds(0, B)], st[3], st[7]).wait()
        pltpu.make_async_copy(q_hbm.at[pl.ds(0, B)], st[4], st[7]).wait()

    def process(b, st, st_n1, st_n2,
                drain_scat_prev, fire_lin2, handle_next):
        d_b, ii_b, jj_b, ri, rj = st[0], st[1], st[2], st[3], st[4]

        def pot_fn(k, _):
            d = d_b[pl.ds(k * 16, 16)]
            x = d * INV_SQRT2_SIGMA
            t = 1.0 / (1.0 + P_C * x)
            poly = t * (A1 + t * (A2 + t * (A3 + t * (A4 + t * A5))))
            pot_buf[pl.ds(k * 16, 16)] = 0.5 * poly * jnp.exp(-x * x) / d
            return 0
        lax.fori_loop(0, B // 16, pot_fn, 0, unroll=4)

        drain_gather(st)

        def mul_fn(v, _):
            pe = v * 4 + iota4
            p = plsc.load_gather(pot_buf, [pe])
            vj = plsc.load_gather(rj, [pe, iotac])
            plsc.store_scatter(rj, [pe, iotac + 4], vj * p)
            plsc.store_scatter(rj, [pe, iotac], zero16)
            vi = plsc.load_gather(ri, [pe, iotac])
            plsc.store_scatter(ri, [pe, iotac + 4], vi * p)
            plsc.store_scatter(ri, [pe, iotac], zero16)
            return 0
        lax.fori_loop(0, (B * N_CH) // 16, mul_fn, 0, unroll=4)

        pltpu.async_copy(rj, tab_sp.at[ii_b], st[7], add=True)
        pltpu.async_copy(ri, tab_sp.at[jj_b], st[7], add=True)

        if drain_scat_prev:
            drain_scat(st_n2)
        if fire_lin2:
            fire_lin(b + 2, st_n2)
        if handle_next:
            drain_lin(b + 1, st_n1)
            fire_gather(st_n1)

    fire_lin(0, sets[0])
    fire_lin(1, sets[1])
    drain_lin(0, sets[0])
    fire_gather(sets[0])

    process(0, sets[0], sets[1], sets[2], False, True, True)

    def group_fn(k, _):
        b = 3 * k + 1
        process(b, sets[1], sets[2], sets[0], True, True, True)
        process(b + 1, sets[2], sets[0], sets[1], True, True, True)
        process(b + 2, sets[0], sets[1], sets[2], True, True, True)
        return 0
    lax.fori_loop(0, (n_blocks - 3) // 3, group_fn, 0)

    process(n_blocks - 2, sets[1], sets[2], sets[0], False, False, True)
    process(n_blocks - 1, sets[2], sets[0], sets[1], False, False, False)

    drain_scat(sets[0])
    drain_scat(sets[1])
    drain_scat(sets[2])

    plsc.subcore_barrier()
    pltpu.sync_copy(tab_sp.at[pl.ds(r0, ROWS_PER_TILE)],
                    out_hbm.at[c].at[pl.ds(r0, ROWS_PER_TILE)])


def _combine_body(p_ref, o_ref):
    o_ref[...] = p_ref[0] + p_ref[1]


def kernel(smearing, charges, neighbor_indices, neighbor_distances):
    del smearing
    e_total = neighbor_distances.shape[0]
    n_groups = max(2, -(-e_total // (NW * B * NBUF)))
    n_blocks = NBUF * n_groups
    e_pad = n_blocks * NW * B
    pad = e_pad - e_total

    idx = neighbor_indices.astype(jnp.int32)
    ii = jnp.concatenate([idx[:, 0], jnp.zeros((pad,), jnp.int32)])
    jj = jnp.concatenate([idx[:, 1], jnp.zeros((pad,), jnp.int32)])
    dist = jnp.concatenate(
        [neighbor_distances.astype(jnp.float32),
         jnp.full((pad,), 1e9, jnp.float32)])
    q = jnp.zeros((N_PAD, D_PAD), jnp.float32)
    q = q.at[:N_NODES, :N_CH].set(charges.astype(jnp.float32))

    mesh = plsc.VectorSubcoreMesh(core_axis_name="c", subcore_axis_name="s",
                                  num_cores=NC, num_subcores=NS)
    buf_types = []
    for _ in range(NBUF):
        buf_types += [
            pltpu.VMEM((B,), jnp.float32),
            pltpu.VMEM((B,), jnp.int32),
            pltpu.VMEM((B,), jnp.int32),
            pltpu.VMEM((B, D_PAD), jnp.float32),
            pltpu.VMEM((B, D_PAD), jnp.float32),
        ]
    sc_call = pl.kernel(
        functools.partial(_sc_body, n_blocks=n_blocks),
        out_type=jax.ShapeDtypeStruct((NC, N_PAD, D_PAD), jnp.float32),
        mesh=mesh,
        compiler_params=pltpu.CompilerParams(use_tc_tiling_on_sc=False,
                                             needs_layout_passes=False),
        scratch_types=buf_types + [
            pltpu.VMEM((B,), jnp.float32),
            pltpu.VMEM_SHARED((N_PAD, D_PAD), jnp.float32),
        ] + [pltpu.SemaphoreType.DMA] * 9,
    )
    partials = sc_call(q, ii, jj, dist)

    flat = partials.reshape(NC, (N_PAD * D_PAD) // 128, 128)
    out = pl.pallas_call(
        _combine_body,
        out_shape=jax.ShapeDtypeStruct(((N_PAD * D_PAD) // 128, 128),
                                       jnp.float32),
    )(flat)
    return out.reshape(N_PAD, D_PAD)[:N_NODES, N_CH:]


# speedup vs baseline: 54.5418x; 1.0060x over previous
import functools
import math

import jax
import jax.numpy as jnp
from jax import lax
from jax.experimental import pallas as pl
from jax.experimental.pallas import tpu as pltpu
from jax.experimental.pallas import tpu_sc as plsc

N_NODES = 100000
N_CH = 4
D_PAD = 8
N_PAD = 100352
SIGMA = 0.2
INV_SQRT2_SIGMA = 1.0 / (math.sqrt(2.0) * SIGMA)

NC = 2
NS = 16
NW = NC * NS

B = 1344
NBUF = 3

P_C = 0.3275911
A1 = 0.254829592
A2 = -0.284496736
A3 = 1.421413741
A4 = -1.453152027
A5 = 1.061405429

ROWS_PER_TILE = N_PAD // NS


def _sc_body(q_hbm, ii_hbm, jj_hbm, d_hbm, out_hbm,
             d0, ii0, jj0, ri0, rj0,
             d1, ii1, jj1, ri1, rj1,
             d2, ii2, jj2, ri2, rj2,
             pot_buf, tab_sp,
             sem_l0, sem_l1, sem_l2,
             sem_g0, sem_g1, sem_g2,
             sem_s0, sem_s1, sem_s2,
             *, n_blocks):
    c = lax.axis_index("c")
    s = lax.axis_index("s")
    tile = c * NS + s

    r0 = s * ROWS_PER_TILE
    pltpu.sync_copy(q_hbm.at[pl.ds(r0, ROWS_PER_TILE)],
                    tab_sp.at[pl.ds(r0, ROWS_PER_TILE)])
    plsc.subcore_barrier()

    iota16 = lax.iota(jnp.int32, 16)
    iota4 = iota16 // 4
    iotac = iota16 % 4
    zero16 = jnp.zeros((16,), jnp.float32)

    edges_per_tile = n_blocks * B

    sets = ((d0, ii0, jj0, ri0, rj0, sem_l0, sem_g0, sem_s0),
            (d1, ii1, jj1, ri1, rj1, sem_l1, sem_g1, sem_s1),
            (d2, ii2, jj2, ri2, rj2, sem_l2, sem_g2, sem_s2))

    def fire_lin(b, st):
        base = tile * edges_per_tile + b * B
        pltpu.async_copy(d_hbm.at[pl.ds(base, B)], st[0], st[5])
        pltpu.async_copy(ii_hbm.at[pl.ds(base, B)], st[1], st[5])
        pltpu.async_copy(jj_hbm.at[pl.ds(base, B)], st[2], st[5])

    def drain_lin(b, st):
        base = tile * edges_per_tile + b * B
        pltpu.make_async_copy(d_hbm.at[pl.ds(base, B)], st[0], st[5]).wait()
        pltpu.make_async_copy(ii_hbm.at[pl.ds(base, B)], st[1], st[5]).wait()
        pltpu.make_async_copy(jj_hbm.at[pl.ds(base, B)], st[2], st[5]).wait()

    def fire_gather(st):
        pltpu.async_copy(tab_sp.at[st[1]], st[3], st[6])
        pltpu.async_copy(tab_sp.at[st[2]], st[4], st[6])

    def drain_gather(st):
        pltpu.make_async_copy(q_hbm.at[pl.ds(0, B)], st[3], st[6]).wait()
        pltpu.make_async_copy(q_hbm.at[pl.ds(0, B)], st[4], st[6]).wait()

    def drain_scat(st):
        pltpu.make_async_copy(q_hbm.at[pl.ds(0, B)], st[3], st[7]).wait()
        pltpu.make_async_copy(q_hbm.at[pl.ds(0, B)], st[4], st[7]).wait()

    def process(b, st, st_n1, st_n2,
                drain_scat_prev, fire_lin2, handle_next):
        d_b, ii_b, jj_b, ri, rj = st[0], st[1], st[2], st[3], st[4]

        def pot_fn(k, _):
            d = d_b[pl.ds(k * 16, 16)]
            x = d * INV_SQRT2_SIGMA
            t = 1.0 / (1.0 + P_C * x)
            poly = t * (A1 + t * (A2 + t * (A3 + t * (A4 + t * A5))))
            pot_buf[pl.ds(k * 16, 16)] = 0.5 * poly * jnp.exp(-x * x) / d
            return 0
        lax.fori_loop(0, B // 16, pot_fn, 0, unroll=8)

        drain_gather(st)

        def mul_fn(v, _):
            pe = v * 4 + iota4
            p = plsc.load_gather(pot_buf, [pe])
            vj = plsc.load_gather(rj, [pe, iotac])
            plsc.store_scatter(rj, [pe, iotac + 4], vj * p)
            plsc.store_scatter(rj, [pe, iotac], zero16)
            vi = plsc.load_gather(ri, [pe, iotac])
            plsc.store_scatter(ri, [pe, iotac + 4], vi * p)
            plsc.store_scatter(ri, [pe, iotac], zero16)
            return 0
        lax.fori_loop(0, (B * N_CH) // 16, mul_fn, 0, unroll=8)

        pltpu.async_copy(rj, tab_sp.at[ii_b], st[7], add=True)
        pltpu.async_copy(ri, tab_sp.at[jj_b], st[7], add=True)

        if drain_scat_prev:
            drain_scat(st_n2)
        if fire_lin2:
            fire_lin(b + 2, st_n2)
        if handle_next:
            drain_lin(b + 1, st_n1)
            fire_gather(st_n1)

    fire_lin(0, sets[0])
    fire_lin(1, sets[1])
    drain_lin(0, sets[0])
    fire_gather(sets[0])

    process(0, sets[0], sets[1], sets[2], False, True, True)

    def group_fn(k, _):
        b = 3 * k + 1
        process(b, sets[1], sets[2], sets[0], True, True, True)
        process(b + 1, sets[2], sets[0], sets[1], True, True, True)
        process(b + 2, sets[0], sets[1], sets[2], True, True, True)
        return 0
    lax.fori_loop(0, (n_blocks - 3) // 3, group_fn, 0)

    process(n_blocks - 2, sets[1], sets[2], sets[0], False, False, True)
    process(n_blocks - 1, sets[2], sets[0], sets[1], False, False, False)

    drain_scat(sets[0])
    drain_scat(sets[1])
    drain_scat(sets[2])

    plsc.subcore_barrier()
    pltpu.sync_copy(tab_sp.at[pl.ds(r0, ROWS_PER_TILE)],
                    out_hbm.at[c].at[pl.ds(r0, ROWS_PER_TILE)])


def _combine_body(p_ref, o_ref):
    o_ref[...] = p_ref[0] + p_ref[1]


def kernel(smearing, charges, neighbor_indices, neighbor_distances):
    del smearing
    e_total = neighbor_distances.shape[0]
    n_groups = max(2, -(-e_total // (NW * B * NBUF)))
    n_blocks = NBUF * n_groups
    e_pad = n_blocks * NW * B
    pad = e_pad - e_total

    idx = neighbor_indices.astype(jnp.int32)
    ii = jnp.concatenate([idx[:, 0], jnp.zeros((pad,), jnp.int32)])
    jj = jnp.concatenate([idx[:, 1], jnp.zeros((pad,), jnp.int32)])
    dist = jnp.concatenate(
        [neighbor_distances.astype(jnp.float32),
         jnp.full((pad,), 1e9, jnp.float32)])
    q = jnp.zeros((N_PAD, D_PAD), jnp.float32)
    q = q.at[:N_NODES, :N_CH].set(charges.astype(jnp.float32))

    mesh = plsc.VectorSubcoreMesh(core_axis_name="c", subcore_axis_name="s",
                                  num_cores=NC, num_subcores=NS)
    buf_types = []
    for _ in range(NBUF):
        buf_types += [
            pltpu.VMEM((B,), jnp.float32),
            pltpu.VMEM((B,), jnp.int32),
            pltpu.VMEM((B,), jnp.int32),
            pltpu.VMEM((B, D_PAD), jnp.float32),
            pltpu.VMEM((B, D_PAD), jnp.float32),
        ]
    sc_call = pl.kernel(
        functools.partial(_sc_body, n_blocks=n_blocks),
        out_type=jax.ShapeDtypeStruct((NC, N_PAD, D_PAD), jnp.float32),
        mesh=mesh,
        compiler_params=pltpu.CompilerParams(use_tc_tiling_on_sc=False,
                                             needs_layout_passes=False),
        scratch_types=buf_types + [
            pltpu.VMEM((B,), jnp.float32),
            pltpu.VMEM_SHARED((N_PAD, D_PAD), jnp.float32),
        ] + [pltpu.SemaphoreType.DMA] * 9,
    )
    partials = sc_call(q, ii, jj, dist)

    flat = partials.reshape(NC, (N_PAD * D_PAD) // 128, 128)
    out = pl.pallas_call(
        _combine_body,
        out_shape=jax.ShapeDtypeStruct(((N_PAD * D_PAD) // 128, 128),
                                       jnp.float32),
    )(flat)
    return out.reshape(N_PAD, D_PAD)[:N_NODES, N_CH:]


# speedup vs baseline: 56.2898x; 1.0320x over previous
import functools
import math

import jax
import jax.numpy as jnp
from jax import lax
from jax.experimental import pallas as pl
from jax.experimental.pallas import tpu as pltpu
from jax.experimental.pallas import tpu_sc as plsc

N_NODES = 100000
N_CH = 4
D_PAD = 8
N_PAD = 100352
SIGMA = 0.2
INV_SQRT2_SIGMA = 1.0 / (math.sqrt(2.0) * SIGMA)

NC = 2
NS = 16
NW = NC * NS

B = 1392
NBUF = 3

P_C = 0.3275911
A1 = 0.254829592
A2 = -0.284496736
A3 = 1.421413741
A4 = -1.453152027
A5 = 1.061405429

ROWS_PER_TILE = N_PAD // NS


def _sc_body(q_hbm, ii_hbm, jj_hbm, d_hbm, out_hbm,
             d0, ii0, jj0, ri0, rj0,
             d1, ii1, jj1, ri1, rj1,
             d2, ii2, jj2, ri2, rj2,
             pot_buf, tab_sp,
             sem_l0, sem_l1, sem_l2,
             sem_g0, sem_g1, sem_g2,
             sem_s0, sem_s1, sem_s2,
             *, n_blocks):
    c = lax.axis_index("c")
    s = lax.axis_index("s")
    tile = c * NS + s

    r0 = s * ROWS_PER_TILE
    pltpu.sync_copy(q_hbm.at[pl.ds(r0, ROWS_PER_TILE)],
                    tab_sp.at[pl.ds(r0, ROWS_PER_TILE)])
    plsc.subcore_barrier()

    iota16 = lax.iota(jnp.int32, 16)
    iota4 = iota16 // 4
    iotac = iota16 % 4
    zero16 = jnp.zeros((16,), jnp.float32)

    edges_per_tile = n_blocks * B

    sets = ((d0, ii0, jj0, ri0, rj0, sem_l0, sem_g0, sem_s0),
            (d1, ii1, jj1, ri1, rj1, sem_l1, sem_g1, sem_s1),
            (d2, ii2, jj2, ri2, rj2, sem_l2, sem_g2, sem_s2))

    def fire_lin(b, st):
        base = tile * edges_per_tile + b * B
        pltpu.async_copy(d_hbm.at[pl.ds(base, B)], st[0], st[5])
        pltpu.async_copy(ii_hbm.at[pl.ds(base, B)], st[1], st[5])
        pltpu.async_copy(jj_hbm.at[pl.ds(base, B)], st[2], st[5])

    def drain_lin(b, st):
        base = tile * edges_per_tile + b * B
        pltpu.make_async_copy(d_hbm.at[pl.ds(base, B)], st[0], st[5]).wait()
        pltpu.make_async_copy(ii_hbm.at[pl.ds(base, B)], st[1], st[5]).wait()
        pltpu.make_async_copy(jj_hbm.at[pl.ds(base, B)], st[2], st[5]).wait()

    def fire_gather(st):
        pltpu.async_copy(tab_sp.at[st[1]], st[3], st[6])
        pltpu.async_copy(tab_sp.at[st[2]], st[4], st[6])

    def drain_gather(st):
        pltpu.make_async_copy(q_hbm.at[pl.ds(0, B)], st[3], st[6]).wait()
        pltpu.make_async_copy(q_hbm.at[pl.ds(0, B)], st[4], st[6]).wait()

    def drain_scat(st):
        pltpu.make_async_copy(q_hbm.at[pl.ds(0, B)], st[3], st[7]).wait()
        pltpu.make_async_copy(q_hbm.at[pl.ds(0, B)], st[4], st[7]).wait()

    def process(b, st, st_n1, st_n2,
                drain_scat_prev, fire_lin2, handle_next):
        d_b, ii_b, jj_b, ri, rj = st[0], st[1], st[2], st[3], st[4]

        def pot_fn(k, _):
            d = d_b[pl.ds(k * 16, 16)]
            x = d * INV_SQRT2_SIGMA
            t = 1.0 / (1.0 + P_C * x)
            poly = t * (A1 + t * (A2 + t * (A3 + t * (A4 + t * A5))))
            pot_buf[pl.ds(k * 16, 16)] = 0.5 * poly * jnp.exp(-x * x) / d
            return 0
        lax.fori_loop(0, B // 16, pot_fn, 0, unroll=8)

        drain_gather(st)

        def mul_fn(v, _):
            pe = v * 4 + iota4
            p = plsc.load_gather(pot_buf, [pe])
            vj = plsc.load_gather(rj, [pe, iotac])
            plsc.store_scatter(rj, [pe, iotac + 4], vj * p)
            plsc.store_scatter(rj, [pe, iotac], zero16)
            vi = plsc.load_gather(ri, [pe, iotac])
            plsc.store_scatter(ri, [pe, iotac + 4], vi * p)
            plsc.store_scatter(ri, [pe, iotac], zero16)
            return 0
        lax.fori_loop(0, (B * N_CH) // 16, mul_fn, 0, unroll=8)

        pltpu.async_copy(rj, tab_sp.at[ii_b], st[7], add=True)
        pltpu.async_copy(ri, tab_sp.at[jj_b], st[7], add=True)

        if drain_scat_prev:
            drain_scat(st_n2)
        if fire_lin2:
            fire_lin(b + 2, st_n2)
        if handle_next:
            drain_lin(b + 1, st_n1)
            fire_gather(st_n1)

    fire_lin(0, sets[0])
    fire_lin(1, sets[1])
    drain_lin(0, sets[0])
    fire_gather(sets[0])

    process(0, sets[0], sets[1], sets[2], False, True, True)

    def group_fn(k, _):
        b = 3 * k + 1
        process(b, sets[1], sets[2], sets[0], True, True, True)
        process(b + 1, sets[2], sets[0], sets[1], True, True, True)
        process(b + 2, sets[0], sets[1], sets[2], True, True, True)
        return 0
    lax.fori_loop(0, (n_blocks - 3) // 3, group_fn, 0)

    process(n_blocks - 2, sets[1], sets[2], sets[0], False, False, True)
    process(n_blocks - 1, sets[2], sets[0], sets[1], False, False, False)

    drain_scat(sets[0])
    drain_scat(sets[1])
    drain_scat(sets[2])

    plsc.subcore_barrier()
    pltpu.sync_copy(tab_sp.at[pl.ds(r0, ROWS_PER_TILE)],
                    out_hbm.at[c].at[pl.ds(r0, ROWS_PER_TILE)])


def _combine_body(p_ref, o_ref):
    o_ref[...] = p_ref[0] + p_ref[1]


def kernel(smearing, charges, neighbor_indices, neighbor_distances):
    del smearing
    e_total = neighbor_distances.shape[0]
    n_groups = max(2, -(-e_total // (NW * B * NBUF)))
    n_blocks = NBUF * n_groups
    e_pad = n_blocks * NW * B
    pad = e_pad - e_total

    idx = neighbor_indices.astype(jnp.int32)
    ii = jnp.concatenate([idx[:, 0], jnp.zeros((pad,), jnp.int32)])
    jj = jnp.concatenate([idx[:, 1], jnp.zeros((pad,), jnp.int32)])
    dist = jnp.concatenate(
        [neighbor_distances.astype(jnp.float32),
         jnp.full((pad,), 1e9, jnp.float32)])
    q = jnp.zeros((N_PAD, D_PAD), jnp.float32)
    q = q.at[:N_NODES, :N_CH].set(charges.astype(jnp.float32))

    mesh = plsc.VectorSubcoreMesh(core_axis_name="c", subcore_axis_name="s",
                                  num_cores=NC, num_subcores=NS)
    buf_types = []
    for _ in range(NBUF):
        buf_types += [
            pltpu.VMEM((B,), jnp.float32),
            pltpu.VMEM((B,), jnp.int32),
            pltpu.VMEM((B,), jnp.int32),
            pltpu.VMEM((B, D_PAD), jnp.float32),
            pltpu.VMEM((B, D_PAD), jnp.float32),
        ]
    sc_call = pl.kernel(
        functools.partial(_sc_body, n_blocks=n_blocks),
        out_type=jax.ShapeDtypeStruct((NC, N_PAD, D_PAD), jnp.float32),
        mesh=mesh,
        compiler_params=pltpu.CompilerParams(use_tc_tiling_on_sc=False,
                                             needs_layout_passes=False),
        scratch_types=buf_types + [
            pltpu.VMEM((B,), jnp.float32),
            pltpu.VMEM_SHARED((N_PAD, D_PAD), jnp.float32),
        ] + [pltpu.SemaphoreType.DMA] * 9,
    )
    partials = sc_call(q, ii, jj, dist)

    flat = partials.reshape(NC, (N_PAD * D_PAD) // 128, 128)
    out = pl.pallas_call(
        _combine_body,
        out_shape=jax.ShapeDtypeStruct(((N_PAD * D_PAD) // 128, 128),
                                       jnp.float32),
    )(flat)
    return out.reshape(N_PAD, D_PAD)[:N_NODES, N_CH:]
